# baseline (speedup 1.0000x reference)
import jax
import jax.numpy as jnp
from jax import lax
from jax.experimental import pallas as pl

_A = 6
_NPAR = 13


def _shift_r(z, d):
    C, L = z.shape
    return jnp.concatenate([jnp.zeros((C, d), z.dtype), z[:, : L - d]], axis=1)


def _shift_l(z, d):
    C, L = z.shape
    return jnp.concatenate([z[:, d:], jnp.zeros((C, d), z.dtype)], axis=1)


def _dwconv3(x, dwv, d):
    return (dwv[:, 0:1] * _shift_r(x, d) + dwv[:, 1:2] * x
            + dwv[:, 2:3] * _shift_l(x, d))


def _ds_conv_grouped(x_parts, dwv, pwv, pbv, dil):
    h = None
    off = 0
    for part in x_parts:
        C = part.shape[0]
        for c0 in range(0, C, 8):
            c1 = min(c0 + 8, C)
            yg = _dwconv3(part[c0:c1, :], dwv[off + c0:off + c1, :], dil)
            hg = jnp.dot(pwv[:, off + c0:off + c1], yg,
                         preferred_element_type=jnp.float32)
            h = hg if h is None else h + hg
        off += C
    return h + pbv


def _block_apply(x_parts, prefs, dil, o_ref=None, staged=False):
    (dw, pw, pb, bng, bnb, cmw, cmb, t1w, t1b, lng, lnb, t2w, t2b) = prefs
    g = bng[...]
    b = bnb[...]
    cmwv = cmw[...]
    cmbv = cmb[...]
    h = jnp.maximum(
        _ds_conv_grouped(x_parts, dw[...], pw[...], pb[...], dil), 0.0)
    C, L = h.shape
    s1 = jnp.sum(h, axis=1, keepdims=True)
    s2 = jnp.sum(h * h, axis=1, keepdims=True)
    m = s1 * (1.0 / L)
    v = jnp.maximum(s2 * (1.0 / L) - m * m, 0.0)
    scale = g * lax.rsqrt(v + 1e-5)
    shift = b - m * scale

    def _tiny(ctx):
        t = jnp.dot(t1w[...], ctx, preferred_element_type=jnp.float32)
        t = t + t1b[...]
        mu = jnp.mean(t)
        var = jnp.mean((t - mu) ** 2)
        t = (t - mu) * lax.rsqrt(var + 1e-5) * lng[...] + lnb[...]
        t = jnp.maximum(t, 0.0)
        return jnp.dot(t2w[...], t, preferred_element_type=jnp.float32) + t2b[...]

    if staged:
        o_ref[...] = h * scale + shift
        xbn = o_ref[...]
        mask = jnp.sum(cmwv * xbn, axis=0, keepdims=True) + cmbv
        mx = jnp.max(mask)
        e = jnp.exp(mask - mx)
        attn = e / jnp.sum(e)
        ctx = jnp.sum(xbn * attn, axis=1, keepdims=True)
        o_ref[...] = o_ref[...] + _tiny(ctx)
        return o_ref
    alpha = cmwv * scale
    beta = cmbv + jnp.sum(cmwv * shift, axis=0, keepdims=True)
    mask = jnp.sum(alpha * h, axis=0, keepdims=True) + beta
    mx = jnp.max(mask)
    e = jnp.exp(mask - mx)
    attn = e / jnp.sum(e)
    hw = jnp.sum(h * attn, axis=1, keepdims=True)
    ctx = scale * (hw - m) + b
    out = h * scale + (shift + _tiny(ctx))
    if o_ref is not None:
        o_ref[...] = out
        return o_ref
    return out


def _block_params_ops(p):
    gc = p['gc']
    cout = p['pw'].shape[0]
    planes = gc['t1_w'].shape[0]
    return [
        p['dw'][:, 0, :],
        p['pw'][:, :, 0],
        p['pb'].reshape(cout, 1),
        p['bn_g'].reshape(cout, 1),
        p['bn_b'].reshape(cout, 1),
        gc['cm_w'].reshape(cout, 1),
        gc['cm_b'].reshape(1, 1),
        gc['t1_w'][:, :, 0],
        gc['t1_b'].reshape(planes, 1),
        gc['ln_g'].reshape(planes, 1),
        gc['ln_b'].reshape(planes, 1),
        gc['t2_w'][:, :, 0],
        gc['t2_b'].reshape(cout, 1),
    ]


def _kernel_a(*refs):
    x_ref = refs[0]
    P = [refs[1 + _NPAR * i: 1 + _NPAR * (i + 1)] for i in range(6)]
    e1o, e2o, e3o, e4o, d1o = refs[1 + _NPAR * 6:]
    e1 = _block_apply([x_ref], P[0], 1, o_ref=e1o)
    e2 = _block_apply([e1], P[1], 1, o_ref=e2o)
    e3 = _block_apply([e2], P[2], 2, o_ref=e3o)
    e4 = _block_apply([e3], P[3], 2, o_ref=e4o)
    e5 = _block_apply([e4], P[4], 3)
    _block_apply([e5], P[5], 3, o_ref=d1o)


def _kernel_b(*refs):
    d1r, e4r, e3r, e2r = refs[:4]
    P = [refs[4 + _NPAR * i: 4 + _NPAR * (i + 1)] for i in range(3)]
    d4o = refs[4 + _NPAR * 3]
    d2 = _block_apply([d1r, e4r], P[0], 2)
    d3 = _block_apply([d2, e3r], P[1], 2)
    _block_apply([d3, e2r], P[2], 1, o_ref=d4o, staged=True)


def _smooth_l1(d):
    ad = jnp.abs(d)
    return jnp.where(ad < 1.0, 0.5 * ad * ad, ad - 0.5)


def _kernel_c(d4r, e1r, *refs):
    P5 = refs[:_NPAR]
    d5o = refs[_NPAR]
    _block_apply([d4r, e1r], P5, 1, o_ref=d5o, staged=True)


def _kernel_d(x_ref, dw, pw, pb, bng, bnb, hw_all, hb_all,
              prob_ref, bbc_ref, bbw_ref):
    h = jnp.maximum(
        _ds_conv_grouped([x_ref], dw[...], pw[...], pb[...], 1), 0.0)
    C, L = h.shape
    s1 = jnp.sum(h, axis=1, keepdims=True)
    s2 = jnp.sum(h * h, axis=1, keepdims=True)
    m = s1 * (1.0 / L)
    v = jnp.maximum(s2 * (1.0 / L) - m * m, 0.0)
    scale = bng[...] * lax.rsqrt(v + 1e-5)
    r = h * scale + (bnb[...] - m * scale)
    z = (jnp.dot(hw_all[...], r, preferred_element_type=jnp.float32)
         + hb_all[...])
    prob_ref[...] = jax.nn.sigmoid(z[0:_A, :])
    bbc_ref[...] = z[_A:2 * _A, :]
    bbw_ref[...] = z[2 * _A:3 * _A, :]


def _kernel_e(prob_ref, bbc_ref, bbw_ref, gt_ref, o_ref):
    prob = prob_ref[...]
    bbc = bbc_ref[...]
    bbw = bbw_ref[...]
    L = prob.shape[1]
    gt = gt_ref[...]
    wvec = jnp.exp2(
        lax.broadcasted_iota(jnp.int32, (_A, 1), 0).astype(jnp.float32) + 3.0)
    pos_i = lax.broadcasted_iota(jnp.int32, (_A, L), 1).astype(jnp.float32)
    a0 = pos_i - wvec * 0.5
    a1 = pos_i + wvec * 0.5
    inside = (a0 >= 0.0) & (a1 < float(L))
    gidx = (lax.broadcasted_iota(jnp.int32, (_A, L), 1) * _A
            + lax.broadcasted_iota(jnp.int32, (_A, L), 0))

    best = jnp.full((_A, L), -1.0, jnp.float32)
    selg0 = jnp.zeros((_A, L), jnp.float32)
    selg1 = jnp.zeros((_A, L), jnp.float32)
    forced = jnp.zeros((_A, L), jnp.bool_)
    for g in range(8):
        g0 = gt[g, 0]
        g1 = gt[g, 1]
        inter = jnp.maximum(0.0, jnp.minimum(a1, g1) - jnp.maximum(a0, g0))
        union = (a1 - a0) + (g1 - g0) - inter
        iou = inter / jnp.maximum(union, 1e-6)
        upd = iou > best
        selg0 = jnp.where(upd, g0, selg0)
        selg1 = jnp.where(upd, g1, selg1)
        best = jnp.where(upd, iou, best)
        ioum = jnp.where(inside, iou, -1.0)
        gmax = jnp.max(ioum)
        cand = jnp.where(ioum == gmax, gidx, jnp.int32(2 ** 30))
        forced = forced | (gidx == jnp.min(cand))

    pos = inside & (forced | (best >= 0.7))
    labeled = inside & (pos | (best < 0.3))
    p = jnp.clip(prob, 1e-7, 1.0 - 1e-7)
    bce = jnp.where(pos, -jnp.log(p), -jnp.log(1.0 - p))
    ce_sum = jnp.sum(jnp.where(labeled, bce, 0.0), axis=(0, 1), keepdims=True)
    n = jnp.sum(labeled.astype(jnp.float32), axis=(0, 1), keepdims=True)
    n_ex = jnp.maximum(n, 1.0)

    aw = wvec + 1.0
    gw = selg1 - selg0 + 1.0
    gctr = selg0 + 0.5 * gw
    t0 = (gctr - (pos_i + 0.5)) / aw
    t1 = jnp.log(gw / aw)
    sl1 = _smooth_l1(bbc - t0) + _smooth_l1(bbw - t1)
    sl_sum = jnp.sum(jnp.where(pos, sl1, 0.0), axis=(0, 1), keepdims=True)

    o_ref[...] = ce_sum / n_ex + sl_sum / n_ex / float(_A * L)


def kernel(sequence, gt_boxes, params):
    x = sequence[0]
    L = x.shape[1]
    f32 = jnp.float32

    enc_p = [_block_params_ops(p) for p in params['enc']]
    dec_p = [_block_params_ops(p) for p in params['dec']]

    ops_a = [x]
    for po in enc_p:
        ops_a += po
    ops_a += dec_p[0]
    e1, e2, e3, e4, d1 = pl.pallas_call(
        _kernel_a,
        out_shape=[jax.ShapeDtypeStruct((c, L), f32)
                   for c in (32, 16, 8, 4, 4)],
    )(*ops_a)

    ops_b = [d1, e4, e3, e2] + dec_p[1] + dec_p[2] + dec_p[3]
    d4 = pl.pallas_call(
        _kernel_b,
        out_shape=jax.ShapeDtypeStruct((32, L), f32),
    )(*ops_b)

    rp = params['rpn']
    hw_all = jnp.concatenate([
        params['cls_w'][:, :, 0],
        params['bbox_w'][0::2, :, 0],
        params['bbox_w'][1::2, :, 0],
    ], axis=0)
    hb_all = jnp.concatenate([
        params['cls_b'],
        params['bbox_b'][0::2],
        params['bbox_b'][1::2],
    ], axis=0).reshape(3 * _A, 1)
    d5 = pl.pallas_call(
        _kernel_c,
        out_shape=jax.ShapeDtypeStruct((32, L), f32),
    )(d4, e1, *dec_p[4])
    prob, bbc, bbw = pl.pallas_call(
        _kernel_d,
        out_shape=[jax.ShapeDtypeStruct((_A, L), f32)] * 3,
    )(d5,
      rp['dw'][:, 0, :],
      rp['pw'][:, :, 0],
      rp['pb'].reshape(-1, 1),
      rp['bn_g'].reshape(-1, 1),
      rp['bn_b'].reshape(-1, 1),
      hw_all, hb_all)
    out = pl.pallas_call(
        _kernel_e,
        out_shape=jax.ShapeDtypeStruct((1, 1), f32),
    )(prob, bbc, bbw, gt_boxes)
    return out[0, 0]


# speedup vs baseline: 1.0264x; 1.0264x over previous
import jax
import jax.numpy as jnp
from jax import lax
from jax.experimental import pallas as pl

_A = 6
_NPAR = 13


def _shift_r(z, d):
    C, L = z.shape
    return jnp.concatenate([jnp.zeros((C, d), z.dtype), z[:, : L - d]], axis=1)


def _shift_l(z, d):
    C, L = z.shape
    return jnp.concatenate([z[:, d:], jnp.zeros((C, d), z.dtype)], axis=1)


def _dwconv3(x, dwv, d):
    return (dwv[:, 0:1] * _shift_r(x, d) + dwv[:, 1:2] * x
            + dwv[:, 2:3] * _shift_l(x, d))


def _ds_conv_grouped(x_parts, dwv, pwv, pbv, dil):
    h = None
    off = 0
    for part in x_parts:
        C = part.shape[0]
        for c0 in range(0, C, 8):
            c1 = min(c0 + 8, C)
            yg = _dwconv3(part[c0:c1, :], dwv[off + c0:off + c1, :], dil)
            hg = jnp.dot(pwv[:, off + c0:off + c1], yg,
                         preferred_element_type=jnp.float32)
            h = hg if h is None else h + hg
        off += C
    return h + pbv


def _block_apply(x_parts, prefs, dil, o_ref=None, staged=False):
    (dw, pw, pb, bng, bnb, cmw, cmb, t1w, t1b, lng, lnb, t2w, t2b) = prefs
    g = bng[...]
    b = bnb[...]
    cmwv = cmw[...]
    cmbv = cmb[...]
    h = jnp.maximum(
        _ds_conv_grouped(x_parts, dw[...], pw[...], pb[...], dil), 0.0)
    C, L = h.shape
    s1 = jnp.sum(h, axis=1, keepdims=True)
    s2 = jnp.sum(h * h, axis=1, keepdims=True)
    m = s1 * (1.0 / L)
    v = jnp.maximum(s2 * (1.0 / L) - m * m, 0.0)
    scale = g * lax.rsqrt(v + 1e-5)
    shift = b - m * scale

    def _tiny(ctx):
        t = jnp.dot(t1w[...], ctx, preferred_element_type=jnp.float32)
        t = t + t1b[...]
        mu = jnp.mean(t)
        var = jnp.mean((t - mu) ** 2)
        t = (t - mu) * lax.rsqrt(var + 1e-5) * lng[...] + lnb[...]
        t = jnp.maximum(t, 0.0)
        return jnp.dot(t2w[...], t, preferred_element_type=jnp.float32) + t2b[...]

    if staged:
        o_ref[...] = (h * scale + shift).astype(o_ref.dtype)
        xbn = o_ref[...]
        mask = jnp.sum(cmwv * xbn, axis=0, keepdims=True) + cmbv
        mx = jnp.max(mask)
        e = jnp.exp(mask - mx)
        attn = e / jnp.sum(e)
        ctx = jnp.sum(xbn * attn, axis=1, keepdims=True)
        o_ref[...] = (o_ref[...] + _tiny(ctx)).astype(o_ref.dtype)
        return o_ref
    alpha = cmwv * scale
    beta = cmbv + jnp.sum(cmwv * shift, axis=0, keepdims=True)
    mask = jnp.sum(alpha * h, axis=0, keepdims=True) + beta
    mx = jnp.max(mask)
    e = jnp.exp(mask - mx)
    attn = e / jnp.sum(e)
    hw = jnp.sum(h * attn, axis=1, keepdims=True)
    ctx = scale * (hw - m) + b
    out = h * scale + (shift + _tiny(ctx))
    if o_ref is not None:
        o_ref[...] = out.astype(o_ref.dtype)
        return o_ref
    return out


def _block_params_ops(p):
    gc = p['gc']
    cout = p['pw'].shape[0]
    planes = gc['t1_w'].shape[0]
    return [
        p['dw'][:, 0, :],
        p['pw'][:, :, 0],
        p['pb'].reshape(cout, 1),
        p['bn_g'].reshape(cout, 1),
        p['bn_b'].reshape(cout, 1),
        gc['cm_w'].reshape(cout, 1),
        gc['cm_b'].reshape(1, 1),
        gc['t1_w'][:, :, 0],
        gc['t1_b'].reshape(planes, 1),
        gc['ln_g'].reshape(planes, 1),
        gc['ln_b'].reshape(planes, 1),
        gc['t2_w'][:, :, 0],
        gc['t2_b'].reshape(cout, 1),
    ]


def _kernel_a(*refs):
    x_ref = refs[0]
    P = [refs[1 + _NPAR * i: 1 + _NPAR * (i + 1)] for i in range(6)]
    e1o, e2o, e3o, e4o, d1o = refs[1 + _NPAR * 6:]
    e1 = _block_apply([x_ref], P[0], 1, o_ref=e1o)
    e2 = _block_apply([e1], P[1], 1, o_ref=e2o)
    e3 = _block_apply([e2], P[2], 2, o_ref=e3o)
    e4 = _block_apply([e3], P[3], 2, o_ref=e4o)
    e5 = _block_apply([e4], P[4], 3)
    _block_apply([e5], P[5], 3, o_ref=d1o)


def _kernel_b(*refs):
    d1r, e4r, e3r, e2r = refs[:4]
    P = [refs[4 + _NPAR * i: 4 + _NPAR * (i + 1)] for i in range(3)]
    d4o = refs[4 + _NPAR * 3]
    d2 = _block_apply([d1r, e4r], P[0], 2)
    d3 = _block_apply([d2, e3r], P[1], 2)
    _block_apply([d3, e2r], P[2], 1, o_ref=d4o, staged=True)


def _smooth_l1(d):
    ad = jnp.abs(d)
    return jnp.where(ad < 1.0, 0.5 * ad * ad, ad - 0.5)


def _kernel_c(d4r, e1r, *refs):
    P5 = refs[:_NPAR]
    d5o = refs[_NPAR]
    _block_apply([d4r, e1r], P5, 1, o_ref=d5o, staged=True)


def _kernel_d(x_ref, dw, pw, pb, bng, bnb, hw_all, hb_all,
              prob_ref, bbc_ref, bbw_ref):
    h = jnp.maximum(
        _ds_conv_grouped([x_ref], dw[...], pw[...], pb[...], 1), 0.0)
    C, L = h.shape
    s1 = jnp.sum(h, axis=1, keepdims=True)
    s2 = jnp.sum(h * h, axis=1, keepdims=True)
    m = s1 * (1.0 / L)
    v = jnp.maximum(s2 * (1.0 / L) - m * m, 0.0)
    scale = bng[...] * lax.rsqrt(v + 1e-5)
    r = h * scale + (bnb[...] - m * scale)
    z = (jnp.dot(hw_all[...], r, preferred_element_type=jnp.float32)
         + hb_all[...])
    prob_ref[...] = jax.nn.sigmoid(z[0:_A, :]).astype(prob_ref.dtype)
    bbc_ref[...] = z[_A:2 * _A, :].astype(bbc_ref.dtype)
    bbw_ref[...] = z[2 * _A:3 * _A, :].astype(bbw_ref.dtype)


def _kernel_e(prob_ref, bbc_ref, bbw_ref, gt_ref, o_ref):
    prob = prob_ref[...].astype(jnp.float32)
    bbc = bbc_ref[...].astype(jnp.float32)
    bbw = bbw_ref[...].astype(jnp.float32)
    L = prob.shape[1]
    gt = gt_ref[...]
    wvec = jnp.exp2(
        lax.broadcasted_iota(jnp.int32, (_A, 1), 0).astype(jnp.float32) + 3.0)
    pos_i = lax.broadcasted_iota(jnp.int32, (_A, L), 1).astype(jnp.float32)
    a0 = pos_i - wvec * 0.5
    a1 = pos_i + wvec * 0.5
    inside = (a0 >= 0.0) & (a1 < float(L))
    gidx = (lax.broadcasted_iota(jnp.int32, (_A, L), 1) * _A
            + lax.broadcasted_iota(jnp.int32, (_A, L), 0))

    best = jnp.full((_A, L), -1.0, jnp.float32)
    selg0 = jnp.zeros((_A, L), jnp.float32)
    selg1 = jnp.zeros((_A, L), jnp.float32)
    forced = jnp.zeros((_A, L), jnp.bool_)
    for g in range(8):
        g0 = gt[g, 0]
        g1 = gt[g, 1]
        inter = jnp.maximum(0.0, jnp.minimum(a1, g1) - jnp.maximum(a0, g0))
        union = (a1 - a0) + (g1 - g0) - inter
        iou = inter / jnp.maximum(union, 1e-6)
        upd = iou > best
        selg0 = jnp.where(upd, g0, selg0)
        selg1 = jnp.where(upd, g1, selg1)
        best = jnp.where(upd, iou, best)
        ioum = jnp.where(inside, iou, -1.0)
        gmax = jnp.max(ioum)
        cand = jnp.where(ioum == gmax, gidx, jnp.int32(2 ** 30))
        forced = forced | (gidx == jnp.min(cand))

    pos = inside & (forced | (best >= 0.7))
    labeled = inside & (pos | (best < 0.3))
    p = jnp.clip(prob, 1e-7, 1.0 - 1e-7)
    bce = jnp.where(pos, -jnp.log(p), -jnp.log(1.0 - p))
    ce_sum = jnp.sum(jnp.where(labeled, bce, 0.0), axis=(0, 1), keepdims=True)
    n = jnp.sum(labeled.astype(jnp.float32), axis=(0, 1), keepdims=True)
    n_ex = jnp.maximum(n, 1.0)

    aw = wvec + 1.0
    gw = selg1 - selg0 + 1.0
    gctr = selg0 + 0.5 * gw
    t0 = (gctr - (pos_i + 0.5)) / aw
    t1 = jnp.log(gw / aw)
    sl1 = _smooth_l1(bbc - t0) + _smooth_l1(bbw - t1)
    sl_sum = jnp.sum(jnp.where(pos, sl1, 0.0), axis=(0, 1), keepdims=True)

    o_ref[...] = ce_sum / n_ex + sl_sum / n_ex / float(_A * L)


def kernel(sequence, gt_boxes, params):
    x = sequence[0]
    L = x.shape[1]
    f32 = jnp.float32

    enc_p = [_block_params_ops(p) for p in params['enc']]
    dec_p = [_block_params_ops(p) for p in params['dec']]

    ops_a = [x]
    for po in enc_p:
        ops_a += po
    ops_a += dec_p[0]
    bf16 = jnp.bfloat16
    e1, e2, e3, e4, d1 = pl.pallas_call(
        _kernel_a,
        out_shape=[jax.ShapeDtypeStruct((c, L), bf16)
                   for c in (32, 16, 8, 4, 4)],
    )(*ops_a)

    ops_b = [d1, e4, e3, e2] + dec_p[1] + dec_p[2] + dec_p[3]
    d4 = pl.pallas_call(
        _kernel_b,
        out_shape=jax.ShapeDtypeStruct((32, L), bf16),
    )(*ops_b)

    rp = params['rpn']
    hw_all = jnp.concatenate([
        params['cls_w'][:, :, 0],
        params['bbox_w'][0::2, :, 0],
        params['bbox_w'][1::2, :, 0],
    ], axis=0)
    hb_all = jnp.concatenate([
        params['cls_b'],
        params['bbox_b'][0::2],
        params['bbox_b'][1::2],
    ], axis=0).reshape(3 * _A, 1)
    d5 = pl.pallas_call(
        _kernel_c,
        out_shape=jax.ShapeDtypeStruct((32, L), bf16),
    )(d4, e1, *dec_p[4])
    prob, bbc, bbw = pl.pallas_call(
        _kernel_d,
        out_shape=[jax.ShapeDtypeStruct((_A, L), bf16)] * 3,
    )(d5,
      rp['dw'][:, 0, :],
      rp['pw'][:, :, 0],
      rp['pb'].reshape(-1, 1),
      rp['bn_g'].reshape(-1, 1),
      rp['bn_b'].reshape(-1, 1),
      hw_all, hb_all)
    out = pl.pallas_call(
        _kernel_e,
        out_shape=jax.ShapeDtypeStruct((1, 1), f32),
    )(prob, bbc, bbw, gt_boxes)
    return out[0, 0]


# speedup vs baseline: 1.0646x; 1.0372x over previous
import jax
import jax.numpy as jnp
from jax import lax
from jax.experimental import pallas as pl

_A = 6
_NPAR = 13


def _shift_r(z, d):
    C, L = z.shape
    return jnp.concatenate([jnp.zeros((C, d), z.dtype), z[:, : L - d]], axis=1)


def _shift_l(z, d):
    C, L = z.shape
    return jnp.concatenate([z[:, d:], jnp.zeros((C, d), z.dtype)], axis=1)


def _dwconv3(x, dwv, d):
    return (dwv[:, 0:1] * _shift_r(x, d) + dwv[:, 1:2] * x
            + dwv[:, 2:3] * _shift_l(x, d))


def _ds_conv_grouped(x_parts, dwv, pwv, pbv, dil):
    h = None
    off = 0
    for part in x_parts:
        C = part.shape[0]
        for c0 in range(0, C, 8):
            c1 = min(c0 + 8, C)
            yg = _dwconv3(part[c0:c1, :], dwv[off + c0:off + c1, :], dil)
            hg = jnp.dot(pwv[:, off + c0:off + c1], yg,
                         preferred_element_type=jnp.float32)
            h = hg if h is None else h + hg
        off += C
    return h + pbv


def _block_apply(x_parts, prefs, dil, o_ref=None, staged=False):
    (dw, pw, pb, bng, bnb, cmw, cmb, t1w, t1b, lng, lnb, t2w, t2b) = prefs
    g = bng[...]
    b = bnb[...]
    cmwv = cmw[...]
    cmbv = cmb[...]
    h = jnp.maximum(
        _ds_conv_grouped(x_parts, dw[...], pw[...], pb[...], dil), 0.0)
    C, L = h.shape
    s1 = jnp.sum(h, axis=1, keepdims=True)
    s2 = jnp.sum(h * h, axis=1, keepdims=True)
    m = s1 * (1.0 / L)
    v = jnp.maximum(s2 * (1.0 / L) - m * m, 0.0)
    scale = g * lax.rsqrt(v + 1e-5)
    shift = b - m * scale

    def _tiny(ctx):
        t = jnp.dot(t1w[...], ctx, preferred_element_type=jnp.float32)
        t = t + t1b[...]
        mu = jnp.mean(t)
        var = jnp.mean((t - mu) ** 2)
        t = (t - mu) * lax.rsqrt(var + 1e-5) * lng[...] + lnb[...]
        t = jnp.maximum(t, 0.0)
        return jnp.dot(t2w[...], t, preferred_element_type=jnp.float32) + t2b[...]

    if staged:
        o_ref[...] = (h * scale + shift).astype(o_ref.dtype)
        xbn = o_ref[...]
        mask = jnp.sum(cmwv * xbn, axis=0, keepdims=True) + cmbv
        mx = jnp.max(mask)
        e = jnp.exp(mask - mx)
        attn = e / jnp.sum(e)
        ctx = jnp.sum(xbn * attn, axis=1, keepdims=True)
        o_ref[...] = (o_ref[...] + _tiny(ctx)).astype(o_ref.dtype)
        return o_ref
    alpha = cmwv * scale
    beta = cmbv + jnp.sum(cmwv * shift, axis=0, keepdims=True)
    mask = jnp.sum(alpha * h, axis=0, keepdims=True) + beta
    mx = jnp.max(mask)
    e = jnp.exp(mask - mx)
    attn = e / jnp.sum(e)
    hw = jnp.sum(h * attn, axis=1, keepdims=True)
    ctx = scale * (hw - m) + b
    out = h * scale + (shift + _tiny(ctx))
    if o_ref is not None:
        o_ref[...] = out.astype(o_ref.dtype)
        return o_ref
    return out


def _block_params_ops(p):
    gc = p['gc']
    cout = p['pw'].shape[0]
    planes = gc['t1_w'].shape[0]
    return [
        p['dw'][:, 0, :],
        p['pw'][:, :, 0],
        p['pb'].reshape(cout, 1),
        p['bn_g'].reshape(cout, 1),
        p['bn_b'].reshape(cout, 1),
        gc['cm_w'].reshape(cout, 1),
        gc['cm_b'].reshape(1, 1),
        gc['t1_w'][:, :, 0],
        gc['t1_b'].reshape(planes, 1),
        gc['ln_g'].reshape(planes, 1),
        gc['ln_b'].reshape(planes, 1),
        gc['t2_w'][:, :, 0],
        gc['t2_b'].reshape(cout, 1),
    ]


def _kernel_a(*refs):
    x_ref = refs[0]
    P = [refs[1 + _NPAR * i: 1 + _NPAR * (i + 1)] for i in range(6)]
    e1o, e2o, e3o, e4o, d1o = refs[1 + _NPAR * 6:]
    e1 = _block_apply([x_ref], P[0], 1, o_ref=e1o)
    e2 = _block_apply([e1], P[1], 1, o_ref=e2o)
    e3 = _block_apply([e2], P[2], 2, o_ref=e3o)
    e4 = _block_apply([e3], P[3], 2, o_ref=e4o)
    e5 = _block_apply([e4], P[4], 3)
    _block_apply([e5], P[5], 3, o_ref=d1o)


def _kernel_b(*refs):
    d1r, e4r, e3r, e2r = refs[:4]
    P = [refs[4 + _NPAR * i: 4 + _NPAR * (i + 1)] for i in range(3)]
    d4o = refs[4 + _NPAR * 3]
    d2 = _block_apply([d1r, e4r], P[0], 2)
    d3 = _block_apply([d2, e3r], P[1], 2)
    _block_apply([d3, e2r], P[2], 1, o_ref=d4o, staged=True)


def _smooth_l1(d):
    ad = jnp.abs(d)
    return jnp.where(ad < 1.0, 0.5 * ad * ad, ad - 0.5)


def _kernel_c(d4r, e1r, *refs):
    P5 = refs[:_NPAR]
    d5o = refs[_NPAR]
    _block_apply([d4r, e1r], P5, 1, o_ref=d5o, staged=True)


def _kernel_f(x_ref, gt_ref, dw, pw, pb, bng, bnb, hw_all, hb_all, o_ref):
    h = jnp.maximum(
        _ds_conv_grouped([x_ref], dw[...], pw[...], pb[...], 1), 0.0)
    C, L = h.shape
    s1 = jnp.sum(h, axis=1, keepdims=True)
    s2 = jnp.sum(h * h, axis=1, keepdims=True)
    m = s1 * (1.0 / L)
    v = jnp.maximum(s2 * (1.0 / L) - m * m, 0.0)
    scale = bng[...] * lax.rsqrt(v + 1e-5)
    r = h * scale + (bnb[...] - m * scale)
    z = (jnp.dot(hw_all[...], r, preferred_element_type=jnp.float32)
         + hb_all[...])
    zc = z[0:_A, :]
    bbc = z[_A:2 * _A, :]
    bbw = z[2 * _A:3 * _A, :]

    gt = gt_ref[...]
    wvec = jnp.exp2(
        lax.broadcasted_iota(jnp.int32, (_A, 1), 0).astype(jnp.float32) + 3.0)
    pos_i = lax.broadcasted_iota(jnp.int32, (_A, L), 1).astype(jnp.float32)
    w2 = wvec * 0.5
    inside = (pos_i - w2 >= 0.0) & (pos_i + w2 < float(L))
    gidx = (lax.broadcasted_iota(jnp.int32, (_A, L), 1) * _A
            + lax.broadcasted_iota(jnp.int32, (_A, L), 0))

    best = jnp.full((_A, L), -1.0, jnp.float32)
    selg0 = jnp.zeros((_A, L), jnp.float32)
    selg1 = jnp.zeros((_A, L), jnp.float32)
    idxs = []
    for g in range(8):
        g0 = gt[g, 0]
        g1 = gt[g, 1]
        a0 = pos_i - w2
        a1 = pos_i + w2
        inter = jnp.maximum(0.0, jnp.minimum(a1, g1) - jnp.maximum(a0, g0))
        union = wvec + (g1 - g0) - inter
        iou = jnp.where(inside, inter / jnp.maximum(union, 1e-6), -1.0)
        upd = iou > best
        selg0 = jnp.where(upd, g0, selg0)
        selg1 = jnp.where(upd, g1, selg1)
        best = jnp.where(upd, iou, best)
        gmax = jnp.max(iou)
        cand = jnp.where(iou == gmax, gidx, jnp.int32(2 ** 30))
        idxs.append(jnp.min(cand))

    forced = (gidx == idxs[0]) | (gidx == idxs[1])
    for g in range(2, 8):
        forced = forced | (gidx == idxs[g])

    pos = inside & (forced | (best >= 0.7))
    labeled = inside & (pos | (best < 0.3))
    cap = 16.11809565095832
    sp_pos = jnp.minimum(jnp.maximum(-zc, 0.0)
                         + jnp.log1p(jnp.exp(-jnp.abs(zc))), cap)
    sp_neg = jnp.minimum(jnp.maximum(zc, 0.0)
                         + jnp.log1p(jnp.exp(-jnp.abs(zc))), cap)
    bce = jnp.where(pos, sp_pos, sp_neg)
    ce_sum = jnp.sum(jnp.where(labeled, bce, 0.0), axis=(0, 1), keepdims=True)
    n = jnp.sum(labeled.astype(jnp.float32), axis=(0, 1), keepdims=True)
    n_ex = jnp.maximum(n, 1.0)

    aw = wvec + 1.0
    gw = selg1 - selg0 + 1.0
    gctr = selg0 + 0.5 * gw
    t0 = (gctr - (pos_i + 0.5)) / aw
    t1 = jnp.log(gw / aw)
    sl1 = _smooth_l1(bbc - t0) + _smooth_l1(bbw - t1)
    sl_sum = jnp.sum(jnp.where(pos, sl1, 0.0), axis=(0, 1), keepdims=True)

    o_ref[...] = ce_sum / n_ex + sl_sum / n_ex / float(_A * L)


def kernel(sequence, gt_boxes, params):
    x = sequence[0]
    L = x.shape[1]
    f32 = jnp.float32

    enc_p = [_block_params_ops(p) for p in params['enc']]
    dec_p = [_block_params_ops(p) for p in params['dec']]

    ops_a = [x]
    for po in enc_p:
        ops_a += po
    ops_a += dec_p[0]
    bf16 = jnp.bfloat16
    e1, e2, e3, e4, d1 = pl.pallas_call(
        _kernel_a,
        out_shape=[jax.ShapeDtypeStruct((c, L), bf16)
                   for c in (32, 16, 8, 4, 4)],
    )(*ops_a)

    ops_b = [d1, e4, e3, e2] + dec_p[1] + dec_p[2] + dec_p[3]
    d4 = pl.pallas_call(
        _kernel_b,
        out_shape=jax.ShapeDtypeStruct((32, L), bf16),
    )(*ops_b)

    rp = params['rpn']
    hw_all = jnp.concatenate([
        params['cls_w'][:, :, 0],
        params['bbox_w'][0::2, :, 0],
        params['bbox_w'][1::2, :, 0],
    ], axis=0)
    hb_all = jnp.concatenate([
        params['cls_b'],
        params['bbox_b'][0::2],
        params['bbox_b'][1::2],
    ], axis=0).reshape(3 * _A, 1)
    d5 = pl.pallas_call(
        _kernel_c,
        out_shape=jax.ShapeDtypeStruct((32, L), bf16),
    )(d4, e1, *dec_p[4])
    out = pl.pallas_call(
        _kernel_f,
        out_shape=jax.ShapeDtypeStruct((1, 1), f32),
    )(d5, gt_boxes,
      rp['dw'][:, 0, :],
      rp['pw'][:, :, 0],
      rp['pb'].reshape(-1, 1),
      rp['bn_g'].reshape(-1, 1),
      rp['bn_b'].reshape(-1, 1),
      hw_all, hb_all)
    return out[0, 0]


# speedup vs baseline: 1.0971x; 1.0305x over previous
import jax
import jax.numpy as jnp
from jax import lax
from jax.experimental import pallas as pl

_A = 6
_NPAR = 14


def _shift_r(z, d):
    C, L = z.shape
    return jnp.concatenate([jnp.zeros((C, d), z.dtype), z[:, : L - d]], axis=1)


def _shift_l(z, d):
    C, L = z.shape
    return jnp.concatenate([z[:, d:], jnp.zeros((C, d), z.dtype)], axis=1)


def _dwconv3(x, dwv, d):
    return (dwv[:, 0:1] * _shift_r(x, d) + dwv[:, 1:2] * x
            + dwv[:, 2:3] * _shift_l(x, d))


def _ds_conv_grouped(x_parts, dwv, pwv, pbv, dil):
    h = None
    off = 0
    for part in x_parts:
        C = part.shape[0]
        for c0 in range(0, C, 8):
            c1 = min(c0 + 8, C)
            yg = _dwconv3(part[c0:c1, :], dwv[off + c0:off + c1, :], dil)
            hg = jnp.dot(pwv[:, off + c0:off + c1], yg,
                         preferred_element_type=jnp.float32)
            h = hg if h is None else h + hg
        off += C
    return h + pbv


def _tap_matmul(x_parts, dwTv, pwv, k):
    z = None
    off = 0
    for part in x_parts:
        C = part.shape[0]
        wk = pwv[:, off:off + C] * dwTv[k:k + 1, off:off + C]
        zp = jnp.dot(wk, part[...] if hasattr(part, 'at') else part,
                     preferred_element_type=jnp.float32)
        z = zp if z is None else z + zp
        off += C
    return z


def _ds_conv_tap(x_parts, dwTv, pwv, pbv, dil):
    h = _tap_matmul(x_parts, dwTv, pwv, 1) + pbv
    h = h + _shift_r(_tap_matmul(x_parts, dwTv, pwv, 0), dil)
    h = h + _shift_l(_tap_matmul(x_parts, dwTv, pwv, 2), dil)
    return h


def _block_apply(x_parts, prefs, dil, o_ref=None, staged=False):
    (dw, pw, pb, bng, bnb, cmw, cmb, t1w, t1b, lng, lnb, t2w, t2b,
     dwT) = prefs
    g = bng[...]
    b = bnb[...]
    cmwv = cmw[...]
    cmbv = cmb[...]
    cin = dw.shape[0]
    cout = pw.shape[0]
    if cout < cin:
        h = _ds_conv_tap(x_parts, dwT[...], pw[...], pb[...], dil)
    else:
        h = _ds_conv_grouped(x_parts, dw[...], pw[...], pb[...], dil)
    h = jnp.maximum(h, 0.0)
    C, L = h.shape
    s1 = jnp.sum(h, axis=1, keepdims=True)
    s2 = jnp.sum(h * h, axis=1, keepdims=True)
    m = s1 * (1.0 / L)
    v = jnp.maximum(s2 * (1.0 / L) - m * m, 0.0)
    scale = g * lax.rsqrt(v + 1e-5)
    shift = b - m * scale

    def _tiny(ctx):
        t = jnp.dot(t1w[...], ctx, preferred_element_type=jnp.float32)
        t = t + t1b[...]
        mu = jnp.mean(t)
        var = jnp.mean((t - mu) ** 2)
        t = (t - mu) * lax.rsqrt(var + 1e-5) * lng[...] + lnb[...]
        t = jnp.maximum(t, 0.0)
        return jnp.dot(t2w[...], t, preferred_element_type=jnp.float32) + t2b[...]

    if staged:
        o_ref[...] = (h * scale + shift).astype(o_ref.dtype)
        xbn = o_ref[...]
        mask = jnp.sum(cmwv * xbn, axis=0, keepdims=True) + cmbv
        mx = jnp.max(mask)
        e = jnp.exp(mask - mx)
        attn = e / jnp.sum(e)
        ctx = jnp.sum(xbn * attn, axis=1, keepdims=True)
        o_ref[...] = (o_ref[...] + _tiny(ctx)).astype(o_ref.dtype)
        return o_ref
    alpha = cmwv * scale
    beta = cmbv + jnp.sum(cmwv * shift, axis=0, keepdims=True)
    mask = jnp.sum(alpha * h, axis=0, keepdims=True) + beta
    mx = jnp.max(mask)
    e = jnp.exp(mask - mx)
    attn = e / jnp.sum(e)
    hw = jnp.sum(h * attn, axis=1, keepdims=True)
    ctx = scale * (hw - m) + b
    out = h * scale + (shift + _tiny(ctx))
    if o_ref is not None:
        o_ref[...] = out.astype(o_ref.dtype)
        return o_ref
    return out


def _block_params_ops(p):
    gc = p['gc']
    cout = p['pw'].shape[0]
    planes = gc['t1_w'].shape[0]
    return [
        p['dw'][:, 0, :],
        p['pw'][:, :, 0],
        p['pb'].reshape(cout, 1),
        p['bn_g'].reshape(cout, 1),
        p['bn_b'].reshape(cout, 1),
        gc['cm_w'].reshape(cout, 1),
        gc['cm_b'].reshape(1, 1),
        gc['t1_w'][:, :, 0],
        gc['t1_b'].reshape(planes, 1),
        gc['ln_g'].reshape(planes, 1),
        gc['ln_b'].reshape(planes, 1),
        gc['t2_w'][:, :, 0],
        gc['t2_b'].reshape(cout, 1),
        p['dw'][:, 0, :].T,
    ]


def _kernel_a(*refs):
    x_ref = refs[0]
    P = [refs[1 + _NPAR * i: 1 + _NPAR * (i + 1)] for i in range(6)]
    e1o, e2o, e3o, e4o, d1o = refs[1 + _NPAR * 6:]
    e1 = _block_apply([x_ref], P[0], 1, o_ref=e1o)
    e2 = _block_apply([e1], P[1], 1, o_ref=e2o)
    e3 = _block_apply([e2], P[2], 2, o_ref=e3o)
    e4 = _block_apply([e3], P[3], 2, o_ref=e4o)
    e5 = _block_apply([e4], P[4], 3)
    _block_apply([e5], P[5], 3, o_ref=d1o)


def _kernel_b(*refs):
    d1r, e4r, e3r, e2r = refs[:4]
    P = [refs[4 + _NPAR * i: 4 + _NPAR * (i + 1)] for i in range(3)]
    d4o = refs[4 + _NPAR * 3]
    d2 = _block_apply([d1r, e4r], P[0], 2)
    d3 = _block_apply([d2, e3r], P[1], 2)
    _block_apply([d3, e2r], P[2], 1, o_ref=d4o, staged=True)


def _smooth_l1(d):
    ad = jnp.abs(d)
    return jnp.where(ad < 1.0, 0.5 * ad * ad, ad - 0.5)


def _kernel_c(d4r, e1r, *refs):
    P5 = refs[:_NPAR]
    d5o = refs[_NPAR]
    _block_apply([d4r, e1r], P5, 1, o_ref=d5o, staged=True)


def _kernel_f(x_ref, gt_ref, dwT, pw, pb, bng, bnb, hw_all, hb_all, o_ref):
    h = jnp.maximum(
        _ds_conv_tap([x_ref], dwT[...], pw[...], pb[...], 1), 0.0)
    C, L = h.shape
    s1 = jnp.sum(h, axis=1, keepdims=True)
    s2 = jnp.sum(h * h, axis=1, keepdims=True)
    m = s1 * (1.0 / L)
    v = jnp.maximum(s2 * (1.0 / L) - m * m, 0.0)
    scale = bng[...] * lax.rsqrt(v + 1e-5)
    r = h * scale + (bnb[...] - m * scale)
    z = (jnp.dot(hw_all[...], r, preferred_element_type=jnp.float32)
         + hb_all[...])
    zc = z[0:_A, :]
    bbc = z[_A:2 * _A, :]
    bbw = z[2 * _A:3 * _A, :]

    gt = gt_ref[...]
    wvec = jnp.exp2(
        lax.broadcasted_iota(jnp.int32, (_A, 1), 0).astype(jnp.float32) + 3.0)
    pos_i = lax.broadcasted_iota(jnp.int32, (_A, L), 1).astype(jnp.float32)
    w2 = wvec * 0.5
    inside = (pos_i - w2 >= 0.0) & (pos_i + w2 < float(L))
    gidx = (lax.broadcasted_iota(jnp.int32, (_A, L), 1) * _A
            + lax.broadcasted_iota(jnp.int32, (_A, L), 0))

    best = jnp.full((_A, L), -1.0, jnp.float32)
    selg0 = jnp.zeros((_A, L), jnp.float32)
    selg1 = jnp.zeros((_A, L), jnp.float32)
    idxs = []
    for g in range(8):
        g0 = gt[g, 0]
        g1 = gt[g, 1]
        a0 = pos_i - w2
        a1 = pos_i + w2
        inter = jnp.maximum(0.0, jnp.minimum(a1, g1) - jnp.maximum(a0, g0))
        union = wvec + (g1 - g0) - inter
        iou = jnp.where(inside, inter / jnp.maximum(union, 1e-6), -1.0)
        upd = iou > best
        selg0 = jnp.where(upd, g0, selg0)
        selg1 = jnp.where(upd, g1, selg1)
        best = jnp.where(upd, iou, best)
        gmax = jnp.max(iou)
        cand = jnp.where(iou == gmax, gidx, jnp.int32(2 ** 30))
        idxs.append(jnp.min(cand))

    forced = (gidx == idxs[0]) | (gidx == idxs[1])
    for g in range(2, 8):
        forced = forced | (gidx == idxs[g])

    pos = inside & (forced | (best >= 0.7))
    labeled = inside & (pos | (best < 0.3))
    cap = 16.11809565095832
    sp_pos = jnp.minimum(jnp.maximum(-zc, 0.0)
                         + jnp.log1p(jnp.exp(-jnp.abs(zc))), cap)
    sp_neg = jnp.minimum(jnp.maximum(zc, 0.0)
                         + jnp.log1p(jnp.exp(-jnp.abs(zc))), cap)
    bce = jnp.where(pos, sp_pos, sp_neg)
    ce_sum = jnp.sum(jnp.where(labeled, bce, 0.0), axis=(0, 1), keepdims=True)
    n = jnp.sum(labeled.astype(jnp.float32), axis=(0, 1), keepdims=True)
    n_ex = jnp.maximum(n, 1.0)

    aw = wvec + 1.0
    gw = selg1 - selg0 + 1.0
    gctr = selg0 + 0.5 * gw
    t0 = (gctr - (pos_i + 0.5)) / aw
    t1 = jnp.log(gw / aw)
    sl1 = _smooth_l1(bbc - t0) + _smooth_l1(bbw - t1)
    sl_sum = jnp.sum(jnp.where(pos, sl1, 0.0), axis=(0, 1), keepdims=True)

    o_ref[...] = ce_sum / n_ex + sl_sum / n_ex / float(_A * L)


def kernel(sequence, gt_boxes, params):
    x = sequence[0]
    L = x.shape[1]
    f32 = jnp.float32

    enc_p = [_block_params_ops(p) for p in params['enc']]
    dec_p = [_block_params_ops(p) for p in params['dec']]

    ops_a = [x]
    for po in enc_p:
        ops_a += po
    ops_a += dec_p[0]
    bf16 = jnp.bfloat16
    e1, e2, e3, e4, d1 = pl.pallas_call(
        _kernel_a,
        out_shape=[jax.ShapeDtypeStruct((c, L), bf16)
                   for c in (32, 16, 8, 4, 4)],
    )(*ops_a)

    ops_b = [d1, e4, e3, e2] + dec_p[1] + dec_p[2] + dec_p[3]
    d4 = pl.pallas_call(
        _kernel_b,
        out_shape=jax.ShapeDtypeStruct((32, L), bf16),
    )(*ops_b)

    rp = params['rpn']
    hw_all = jnp.concatenate([
        params['cls_w'][:, :, 0],
        params['bbox_w'][0::2, :, 0],
        params['bbox_w'][1::2, :, 0],
    ], axis=0)
    hb_all = jnp.concatenate([
        params['cls_b'],
        params['bbox_b'][0::2],
        params['bbox_b'][1::2],
    ], axis=0).reshape(3 * _A, 1)
    d5 = pl.pallas_call(
        _kernel_c,
        out_shape=jax.ShapeDtypeStruct((32, L), bf16),
    )(d4, e1, *dec_p[4])
    out = pl.pallas_call(
        _kernel_f,
        out_shape=jax.ShapeDtypeStruct((1, 1), f32),
    )(d5, gt_boxes,
      rp['dw'][:, 0, :].T,
      rp['pw'][:, :, 0],
      rp['pb'].reshape(-1, 1),
      rp['bn_g'].reshape(-1, 1),
      rp['bn_b'].reshape(-1, 1),
      hw_all, hb_all)
    return out[0, 0]
